# baseline (speedup 1.0000x reference)
import functools

import jax
import jax.numpy as jnp
from jax import lax
from jax.experimental import pallas as pl
from jax.experimental.pallas import tpu as pltpu
from jax.experimental.pallas import tpu_sc as plsc

_BT = 1024

_NC = 2
_NS = 16
_NW = _NC * _NS
_CHUNK = 16


def _scores_body(x_ref, w1_ref, b1_ref, w2_ref, b2_ref, idx_ref):
    x = x_ref[...]
    h = jnp.dot(x, w1_ref[...], preferred_element_type=jnp.float32)
    h = jnp.maximum(h + b1_ref[...], 0.0)
    s = jnp.dot(h, w2_ref[...], preferred_element_type=jnp.float32) + b2_ref[...]
    num_e = s.shape[1]
    m = jnp.max(s, axis=1, keepdims=True)
    ii = lax.broadcasted_iota(jnp.int32, s.shape, 1)
    idx_ref[...] = jnp.min(jnp.where(s == m, ii, num_e), axis=1)


def _tc_indices(flat, W1, b1, W2, b2):
    N, H = flat.shape
    RH = W1.shape[1]
    E = W2.shape[1]
    return pl.pallas_call(
        _scores_body,
        grid=(N // _BT,),
        in_specs=[
            pl.BlockSpec((_BT, H), lambda i: (i, 0)),
            pl.BlockSpec((H, RH), lambda i: (0, 0)),
            pl.BlockSpec((1, RH), lambda i: (0, 0)),
            pl.BlockSpec((RH, E), lambda i: (0, 0)),
            pl.BlockSpec((1, E), lambda i: (0, 0)),
        ],
        out_specs=pl.BlockSpec((_BT,), lambda i: (i,)),
        out_shape=jax.ShapeDtypeStruct((N,), jnp.int32),
    )(flat, W1, b1.reshape(1, RH), W2, b2.reshape(1, E))


def _affine_body(x_hbm, idx_hbm, es_hbm, eb_hbm, out_hbm,
                 idx_v, x_v, es_v, out_v, sem_es, sem_eb):
    wid = lax.axis_index("s") * _NC + lax.axis_index("c")
    n_tok = x_hbm.shape[0]
    tok_per_w = n_tok // _NW
    base = wid * tok_per_w
    n_chunks = tok_per_w // _CHUNK
    n_cols = x_hbm.shape[1] // 16

    def chunk_body(c, carry):
        t0 = base + c * _CHUNK
        pltpu.sync_copy(idx_hbm.at[pl.ds(t0, _CHUNK)], idx_v)
        eb_cp = pltpu.async_copy(eb_hbm.at[idx_v], out_v, sem_eb)
        es_cp = pltpu.async_copy(es_hbm.at[idx_v], es_v, sem_es)
        pltpu.sync_copy(x_hbm.at[pl.ds(t0, _CHUNK), :], x_v)
        eb_cp.wait()
        es_cp.wait()

        def col_body(j, carry2):
            for t in range(_CHUNK):
                sl = (t, pl.ds(j * 16, 16))
                out_v[sl] = out_v[sl] + x_v[sl] * es_v[sl]
            return carry2

        lax.fori_loop(0, n_cols, col_body, 0, unroll=False)
        pltpu.sync_copy(out_v, out_hbm.at[pl.ds(t0, _CHUNK), :])
        return carry

    lax.fori_loop(0, n_chunks, chunk_body, 0, unroll=False)


def _sc_affine(flat, idx, expert_scales, expert_biases):
    N, H = flat.shape
    mesh = plsc.VectorSubcoreMesh(core_axis_name="c", subcore_axis_name="s")
    kern = functools.partial(
        pl.kernel, mesh=mesh,
        out_type=jax.ShapeDtypeStruct((N, H), jnp.float32),
        scratch_types=[
            pltpu.VMEM((_CHUNK,), jnp.int32),
            pltpu.VMEM((_CHUNK, H), jnp.float32),
            pltpu.VMEM((_CHUNK, H), jnp.float32),
            pltpu.VMEM((_CHUNK, H), jnp.float32),
            pltpu.SemaphoreType.DMA,
            pltpu.SemaphoreType.DMA,
        ],
    )(_affine_body)
    return kern(flat, idx, expert_scales, expert_biases)


def kernel(hidden_states, W1, b1, W2, b2, Wc, bc, Wg1, bg1, Wg2, bg2,
           expert_scales, expert_biases):
    B, S, H = hidden_states.shape
    N = B * S
    flat = hidden_states.reshape(N, H)
    idx = _tc_indices(flat, W1, b1, W2, b2)
    out = _sc_affine(flat, idx, expert_scales, expert_biases)
    return out.reshape(B, S, H)
